# baseline (speedup 1.0000x reference)
import functools

import jax
import jax.numpy as jnp
import numpy as np
from jax import lax
from jax.experimental import pallas as pl
from jax.experimental.pallas import tpu as pltpu
from jax.experimental.pallas import tpu_sc as plsc

_NC = 2
_NS = 16
_L = 16
_W = _NC * _NS
_CH = 64

_HMASK = np.uint32(0xFFFF0000)


def _rne_bf16_bits(f32_bits):
    return f32_bits + np.uint32(0x7FFF) + ((f32_bits >> 16) & np.uint32(1))


def _pack_halves_u32(a, b):
    ab = lax.bitcast_convert_type(a, jnp.uint32)
    bb = lax.bitcast_convert_type(b, jnp.uint32)
    return (_rne_bf16_bits(ab) >> 16) | (_rne_bf16_bits(bb) & _HMASK)


def _sc_edge_pass(xp, src4, dst4, ee5, npad, h):
    n, hw = xp.shape
    _, nchb, _, ch = src4.shape
    rpt = npad // _NS
    nz = rpt // ch

    mesh = plsc.VectorSubcoreMesh(core_axis_name="c", subcore_axis_name="s")

    @functools.partial(
        pl.kernel,
        out_type=jax.ShapeDtypeStruct((_NC, npad, h), jnp.float32),
        mesh=mesh,
        compiler_params=pltpu.CompilerParams(needs_layout_passes=False,
                                             use_tc_tiling_on_sc=False),
        scratch_types=[
            pltpu.VMEM((2, 8, ch), jnp.int32),
            pltpu.VMEM((2, 8, ch), jnp.int32),
            pltpu.VMEM((2, ch, hw), jnp.uint32),
            pltpu.VMEM((2, ch, hw), jnp.uint32),
            pltpu.VMEM((2, ch, h), jnp.float32),
            pltpu.VMEM_SHARED((npad, h), jnp.float32),
            pltpu.SemaphoreType.DMA,
            pltpu.SemaphoreType.DMA,
            pltpu.SemaphoreType.DMA,
            pltpu.SemaphoreType.DMA,
            pltpu.SemaphoreType.DMA,
            pltpu.SemaphoreType.DMA,
            pltpu.SemaphoreType.DMA,
        ],
    )
    def k(x_hbm, src_hbm, dst_hbm, ee_hbm, out_hbm,
          src_v, dst_v, xg_v, ee_v, msg_v, agg_sh,
          sem_i, sem_g0, sem_g1, sem_e0, sem_e1, sem_s0, sem_s1):
        cid = lax.axis_index("c")
        sid = lax.axis_index("s")
        wid = cid * _NS + sid
        sem_g = (sem_g0, sem_g1)
        sem_e = (sem_e0, sem_e1)
        sem_s = (sem_s0, sem_s1)

        zero = jnp.zeros((_L,), jnp.float32)

        def zero_body(i, _):
            for j in range(h // _L):
                msg_v[0, i, pl.ds(j * _L, _L)] = zero
            return 0

        lax.fori_loop(0, ch, zero_body, 0)
        for r in range(nz):
            pltpu.sync_copy(msg_v.at[0],
                            agg_sh.at[pl.ds(sid * rpt + r * ch, ch)])
        plsc.subcore_barrier()

        def compute_chunk(p):
            def body(i, _):
                for g in range(hw // _L):
                    s = pl.ds(g * _L, _L)
                    xw = xg_v[p, i, s]
                    ew = ee_v[p, i, s]
                    xlo = plsc.bitcast(xw << 16, jnp.float32)
                    xhi = plsc.bitcast(xw & _HMASK, jnp.float32)
                    elo = plsc.bitcast(ew << 16, jnp.float32)
                    ehi = plsc.bitcast(ew & _HMASK, jnp.float32)
                    msg_v[p, i, pl.ds(g * _L, _L)] = (
                        jnp.maximum(xlo + elo, 0.0))
                    msg_v[p, i, pl.ds(hw + g * _L, _L)] = (
                        jnp.maximum(xhi + ehi, 0.0))
                return 0
            lax.fori_loop(0, ch, body, 0)

        def drain(dst_ref, sem, f32src=False):
            src = out_hbm.at[0, pl.ds(0, ch)] if f32src else ee_hbm.at[wid, 0, 0]
            pltpu.make_async_copy(src, dst_ref, sem).wait()

        def emit_block(jb, q, first, last):
            if not last:
                ci_s = pltpu.async_copy(src_hbm.at[wid, jb + 1],
                                        src_v.at[1 - q], sem_i)
                ci_d = pltpu.async_copy(dst_hbm.at[wid, jb + 1],
                                        dst_v.at[1 - q], sem_i)
            for jj in range(8):
                p = jj & 1
                drain(xg_v.at[p], sem_g[p])
                drain(ee_v.at[p], sem_e[p])
                if not (first and jj == 0):
                    drain(msg_v.at[1 - p], sem_s[1 - p], f32src=True)
                if jj < 7:
                    pltpu.async_copy(x_hbm.at[src_v.at[q, jj + 1]],
                                     xg_v.at[1 - p], sem_g[1 - p])
                    pltpu.async_copy(ee_hbm.at[wid, jb, jj + 1],
                                     ee_v.at[1 - p], sem_e[1 - p])
                elif not last:
                    ci_s.wait()
                    ci_d.wait()
                    pltpu.async_copy(x_hbm.at[src_v.at[1 - q, 0]],
                                     xg_v.at[1 - p], sem_g[1 - p])
                    pltpu.async_copy(ee_hbm.at[wid, jb + 1, 0],
                                     ee_v.at[1 - p], sem_e[1 - p])
                compute_chunk(p)
                pltpu.async_copy(msg_v.at[p], agg_sh.at[dst_v.at[q, jj]],
                                 sem_s[p], add=True)
            return None

        pltpu.sync_copy(src_hbm.at[wid, 0], src_v.at[0])
        pltpu.sync_copy(dst_hbm.at[wid, 0], dst_v.at[0])
        pltpu.async_copy(x_hbm.at[src_v.at[0, 0]], xg_v.at[0], sem_g[0])
        pltpu.async_copy(ee_hbm.at[wid, 0, 0], ee_v.at[0], sem_e[0])

        emit_block(0, 0, True, False)

        def blk_body(jb, _):
            emit_block(jb, jb & 1, False, False)
            return 0

        lax.fori_loop(1, nchb - 1, blk_body, 0)
        emit_block(nchb - 1, (nchb - 1) & 1, False, True)
        drain(msg_v.at[1], sem_s[1], f32src=True)

        plsc.subcore_barrier()
        for r in range(nz):
            rows = pl.ds(sid * rpt + r * ch, ch)
            pltpu.sync_copy(agg_sh.at[rows], out_hbm.at[cid, rows])

    return k(xp, src4, dst4, ee5)


def _edge_mlp(ea, we1, bee1, we2, bee2):
    e, ed = ea.shape
    h = we1.shape[1]
    hw = h // 2
    be = 4096
    grid = e // be

    def body(ea_ref, w1a_ref, b1a_ref, w1b_ref, b1b_ref,
             w2a_ref, b2a_ref, w2b_ref, b2b_ref, o1_ref, o2_ref):
        a = ea_ref[...]

        def packed(wa, ba, wb, bb):
            lo = jnp.dot(a, wa[...], preferred_element_type=jnp.float32) + ba[...]
            hi = jnp.dot(a, wb[...], preferred_element_type=jnp.float32) + bb[...]
            return _pack_halves_u32(lo, hi)

        o1_ref[...] = packed(w1a_ref, b1a_ref, w1b_ref, b1b_ref)
        o2_ref[...] = packed(w2a_ref, b2a_ref, w2b_ref, b2b_ref)

    wspec = pl.BlockSpec((ed, hw), lambda i: (0, 0))
    bspec = pl.BlockSpec((1, hw), lambda i: (0, 0))
    return pl.pallas_call(
        body,
        grid=(grid,),
        in_specs=[pl.BlockSpec((be, ed), lambda i: (i, 0)),
                  wspec, bspec, wspec, bspec, wspec, bspec, wspec, bspec],
        out_specs=[pl.BlockSpec((be, hw), lambda i: (i, 0))] * 2,
        out_shape=[jax.ShapeDtypeStruct((e, hw), jnp.uint32)] * 2,
    )(ea,
      we1[:, :hw], bee1[:hw].reshape(1, hw), we1[:, hw:], bee1[hw:].reshape(1, hw),
      we2[:, :hw], bee2[:hw].reshape(1, hw), we2[:, hw:], bee2[hw:].reshape(1, hw))


def _bn_rows(hh, g, bt):
    m = jnp.mean(hh, axis=0, keepdims=True)
    v = jnp.mean((hh - m) ** 2, axis=0, keepdims=True)
    return (hh - m) / jnp.sqrt(v + 1e-5) * g + bt


def _node_update1(x, agg, wn, bnn, g, bt):
    n, h = x.shape
    hw = h // 2

    def body(x_ref, a_ref, wa_ref, ba_ref, ga_ref, ta_ref,
             wb_ref, bb_ref, gb_ref, tb_ref, o_ref, op_ref):
        t = x_ref[...] + a_ref[0, :n] + a_ref[1, :n]
        ha = jnp.dot(t, wa_ref[...], preferred_element_type=jnp.float32)
        ha = jnp.maximum(_bn_rows(ha + ba_ref[...], ga_ref[...], ta_ref[...]),
                         0.0)
        hb = jnp.dot(t, wb_ref[...], preferred_element_type=jnp.float32)
        hb = jnp.maximum(_bn_rows(hb + bb_ref[...], gb_ref[...], tb_ref[...]),
                         0.0)
        o_ref[:, :hw] = ha
        o_ref[:, hw:] = hb
        op_ref[...] = _pack_halves_u32(ha, hb)

    return pl.pallas_call(
        body,
        out_shape=[jax.ShapeDtypeStruct((n, h), jnp.float32),
                   jax.ShapeDtypeStruct((n, hw), jnp.uint32)],
    )(x, agg,
      wn[:, :hw], bnn[:hw].reshape(1, hw), g[:hw].reshape(1, hw),
      bt[:hw].reshape(1, hw),
      wn[:, hw:], bnn[hw:].reshape(1, hw), g[hw:].reshape(1, hw),
      bt[hw:].reshape(1, hw))


def _node_update2_head(h1, agg, wn2, bnn2, g2, bt2,
                       wa1, ba1, ga1, bta1, wa2, ba2, ga2, bta2, wa3, ba3):
    n, h = h1.shape
    a = wa3.shape[1]

    def body(h1_ref, a_ref, wn_ref, bn_ref, g2_ref, t2_ref,
             w1_ref, b1_ref, g1_ref, t1_ref,
             w2_ref, b2_ref, gg2_ref, tt2_ref,
             w3_ref, b3_ref, o_ref):
        t = h1_ref[...] + a_ref[0, :n] + a_ref[1, :n]
        hh = jnp.dot(t, wn_ref[...], preferred_element_type=jnp.float32)
        hh = jnp.maximum(_bn_rows(hh + bn_ref[...], g2_ref[...], t2_ref[...]), 0.0)
        hh = jax.nn.sigmoid(hh)
        z1 = jnp.dot(hh, w1_ref[...], preferred_element_type=jnp.float32)
        z1 = jnp.maximum(_bn_rows(z1 + b1_ref[...], g1_ref[...], t1_ref[...]), 0.0)
        z2 = jnp.dot(z1, w2_ref[...], preferred_element_type=jnp.float32)
        z2 = jnp.maximum(_bn_rows(z2 + b2_ref[...], gg2_ref[...], tt2_ref[...]), 0.0)
        z3 = jnp.dot(z2, w3_ref[...], preferred_element_type=jnp.float32)
        o_ref[...] = jax.nn.sigmoid(z3 + b3_ref[...])

    return pl.pallas_call(
        body,
        out_shape=jax.ShapeDtypeStruct((n, a), jnp.float32),
    )(h1, agg, wn2, bnn2.reshape(1, h), g2.reshape(1, h), bt2.reshape(1, h),
      wa1, ba1.reshape(1, h), ga1.reshape(1, h), bta1.reshape(1, h),
      wa2, ba2.reshape(1, h), ga2.reshape(1, h), bta2.reshape(1, h),
      wa3, ba3.reshape(1, a))


def kernel(x, edge_index, edge_attr, batch, we1, bee1, wn1, bnn1, g1, bt1,
           we2, bee2, wn2, bnn2, g2, bt2, wa1, ba1, ga1, bta1,
           wa2, ba2, ga2, bta2, wa3, ba3):
    n, h = x.shape
    hw = h // 2
    e = edge_attr.shape[0]
    nchb = -(-e // (_W * 8 * _CH))
    epad = _W * nchb * 8 * _CH
    npad = -(-n // (_NS * _CH)) * (_NS * _CH)

    src_p = jnp.concatenate(
        [edge_index[0].astype(jnp.int32),
         jnp.arange(epad - e, dtype=jnp.int32) % n])
    dst_p = jnp.concatenate(
        [edge_index[1].astype(jnp.int32),
         n + jnp.arange(epad - e, dtype=jnp.int32) % (npad - n)])
    src4 = src_p.reshape(_W, nchb, 8, _CH)
    dst4 = dst_p.reshape(_W, nchb, 8, _CH)
    ea_p = jnp.concatenate(
        [edge_attr, jnp.zeros((epad - e, edge_attr.shape[1]), jnp.float32)])

    x_p = _pack_halves_u32(x[:, :hw], x[:, hw:])
    ee1, ee2 = _edge_mlp(ea_p, we1, bee1, we2, bee2)
    ee1_5 = ee1.reshape(_W, nchb, 8, _CH, hw)
    ee2_5 = ee2.reshape(_W, nchb, 8, _CH, hw)

    agg1 = _sc_edge_pass(x_p, src4, dst4, ee1_5, npad, h)
    h1, h1_p = _node_update1(x, agg1, wn1, bnn1, g1, bt1)
    agg2 = _sc_edge_pass(h1_p, src4, dst4, ee2_5, npad, h)
    return _node_update2_head(h1, agg2, wn2, bnn2, g2, bt2,
                              wa1, ba1, ga1, bta1, wa2, ba2, ga2, bta2,
                              wa3, ba3)


# speedup vs baseline: 1.5887x; 1.5887x over previous
import functools

import jax
import jax.numpy as jnp
from jax import lax
from jax.experimental import pallas as pl
from jax.experimental.pallas import tpu as pltpu
from jax.experimental.pallas import tpu_sc as plsc

_NC = 2
_NS = 16
_L = 16
_W = _NC * _NS
_CH = 80


def _sc_edge_pass(xn, src4, dst4, ee5, npad):
    n, h = xn.shape
    _, nchb, _, ch = src4.shape
    rpt = npad // _NS
    nz = rpt // ch

    mesh = plsc.VectorSubcoreMesh(core_axis_name="c", subcore_axis_name="s")

    @functools.partial(
        pl.kernel,
        out_type=jax.ShapeDtypeStruct((_NC, npad, h), jnp.float32),
        mesh=mesh,
        scratch_types=[
            pltpu.VMEM((2, 8, ch), jnp.int32),
            pltpu.VMEM((2, 8, ch), jnp.int32),
            pltpu.VMEM((2, ch, h), jnp.float32),
            pltpu.VMEM((2, ch, h), jnp.float32),
            pltpu.VMEM_SHARED((npad, h), jnp.float32),
            pltpu.SemaphoreType.DMA,
            pltpu.SemaphoreType.DMA,
            pltpu.SemaphoreType.DMA,
            pltpu.SemaphoreType.DMA,
            pltpu.SemaphoreType.DMA,
            pltpu.SemaphoreType.DMA,
            pltpu.SemaphoreType.DMA,
        ],
    )
    def k(x_hbm, src_hbm, dst_hbm, ee_hbm, out_hbm,
          src_v, dst_v, xg_v, ee_v, agg_sh,
          sem_i, sem_g0, sem_g1, sem_e0, sem_e1, sem_s0, sem_s1):
        cid = lax.axis_index("c")
        sid = lax.axis_index("s")
        wid = cid * _NS + sid
        sem_g = (sem_g0, sem_g1)
        sem_e = (sem_e0, sem_e1)
        sem_s = (sem_s0, sem_s1)

        zero = jnp.zeros((_L,), jnp.float32)

        def zero_body(i, _):
            for j in range(h // _L):
                xg_v[0, i, pl.ds(j * _L, _L)] = zero
            return 0

        lax.fori_loop(0, ch, zero_body, 0)
        for r in range(nz):
            pltpu.sync_copy(xg_v.at[0],
                            agg_sh.at[pl.ds(sid * rpt + r * ch, ch)])
        plsc.subcore_barrier()

        def compute_chunk(p):
            def body(i, _):
                for j in range(h // _L):
                    s = pl.ds(j * _L, _L)
                    xg_v[p, i, s] = jnp.maximum(xg_v[p, i, s] + ee_v[p, i, s],
                                                0.0)
                return 0
            lax.fori_loop(0, ch, body, 0)

        def drain(dst_ref, sem):
            pltpu.make_async_copy(ee_hbm.at[wid, 0, 0], dst_ref, sem).wait()

        def emit_block(jb, q, first, last):
            if not last:
                ci_s = pltpu.async_copy(src_hbm.at[wid, jb + 1],
                                        src_v.at[1 - q], sem_i)
                ci_d = pltpu.async_copy(dst_hbm.at[wid, jb + 1],
                                        dst_v.at[1 - q], sem_i)
            for jj in range(8):
                p = jj & 1
                drain(xg_v.at[p], sem_g[p])
                drain(ee_v.at[p], sem_e[p])
                if not (first and jj == 0):
                    drain(xg_v.at[1 - p], sem_s[1 - p])
                if jj < 7:
                    pltpu.async_copy(x_hbm.at[src_v.at[q, jj + 1]],
                                     xg_v.at[1 - p], sem_g[1 - p])
                    pltpu.async_copy(ee_hbm.at[wid, jb, jj + 1],
                                     ee_v.at[1 - p], sem_e[1 - p])
                elif not last:
                    ci_s.wait()
                    ci_d.wait()
                    pltpu.async_copy(x_hbm.at[src_v.at[1 - q, 0]],
                                     xg_v.at[1 - p], sem_g[1 - p])
                    pltpu.async_copy(ee_hbm.at[wid, jb + 1, 0],
                                     ee_v.at[1 - p], sem_e[1 - p])
                compute_chunk(p)
                pltpu.async_copy(xg_v.at[p], agg_sh.at[dst_v.at[q, jj]],
                                 sem_s[p], add=True)
            return None

        pltpu.sync_copy(src_hbm.at[wid, 0], src_v.at[0])
        pltpu.sync_copy(dst_hbm.at[wid, 0], dst_v.at[0])
        pltpu.async_copy(x_hbm.at[src_v.at[0, 0]], xg_v.at[0], sem_g[0])
        pltpu.async_copy(ee_hbm.at[wid, 0, 0], ee_v.at[0], sem_e[0])

        emit_block(0, 0, True, False)

        def blk_body(jb, _):
            emit_block(jb, jb & 1, False, False)
            return 0

        lax.fori_loop(1, nchb - 1, blk_body, 0)
        emit_block(nchb - 1, (nchb - 1) & 1, False, True)
        drain(xg_v.at[1], sem_s[1])

        plsc.subcore_barrier()
        for r in range(nz):
            rows = pl.ds(sid * rpt + r * ch, ch)
            pltpu.sync_copy(agg_sh.at[rows], out_hbm.at[cid, rows])

    return k(xn, src4, dst4, ee5)


def _edge_mlp(ea, we, bee):
    e, ed = ea.shape
    h = we.shape[1]
    be = 4096
    grid = e // be

    def body(ea_ref, w_ref, b_ref, o_ref):
        o_ref[...] = jnp.dot(ea_ref[...], w_ref[...],
                             preferred_element_type=jnp.float32) + b_ref[...]

    return pl.pallas_call(
        body,
        grid=(grid,),
        in_specs=[
            pl.BlockSpec((be, ed), lambda i: (i, 0)),
            pl.BlockSpec((ed, h), lambda i: (0, 0)),
            pl.BlockSpec((1, h), lambda i: (0, 0)),
        ],
        out_specs=pl.BlockSpec((be, h), lambda i: (i, 0)),
        out_shape=jax.ShapeDtypeStruct((e, h), jnp.float32),
    )(ea, we, bee.reshape(1, h))


def _bn_rows(hh, g, bt):
    m = jnp.mean(hh, axis=0, keepdims=True)
    v = jnp.mean((hh - m) ** 2, axis=0, keepdims=True)
    return (hh - m) / jnp.sqrt(v + 1e-5) * g + bt


def _node_update1(x, agg, wn, bnn, g, bt):
    n, h = x.shape

    def body(x_ref, a_ref, w_ref, b_ref, g_ref, t_ref, o_ref):
        t = x_ref[...] + a_ref[0, :n] + a_ref[1, :n]
        hh = jnp.dot(t, w_ref[...], preferred_element_type=jnp.float32)
        hh = _bn_rows(hh + b_ref[...], g_ref[...], t_ref[...])
        o_ref[...] = jnp.maximum(hh, 0.0)

    return pl.pallas_call(
        body,
        out_shape=jax.ShapeDtypeStruct((n, h), jnp.float32),
    )(x, agg, wn, bnn.reshape(1, h), g.reshape(1, h), bt.reshape(1, h))


def _node_update2_head(h1, agg, wn2, bnn2, g2, bt2,
                       wa1, ba1, ga1, bta1, wa2, ba2, ga2, bta2, wa3, ba3):
    n, h = h1.shape
    a = wa3.shape[1]

    def body(h1_ref, a_ref, wn_ref, bn_ref, g2_ref, t2_ref,
             w1_ref, b1_ref, g1_ref, t1_ref,
             w2_ref, b2_ref, gg2_ref, tt2_ref,
             w3_ref, b3_ref, o_ref):
        t = h1_ref[...] + a_ref[0, :n] + a_ref[1, :n]
        hh = jnp.dot(t, wn_ref[...], preferred_element_type=jnp.float32)
        hh = jnp.maximum(_bn_rows(hh + bn_ref[...], g2_ref[...], t2_ref[...]), 0.0)
        hh = jax.nn.sigmoid(hh)
        z1 = jnp.dot(hh, w1_ref[...], preferred_element_type=jnp.float32)
        z1 = jnp.maximum(_bn_rows(z1 + b1_ref[...], g1_ref[...], t1_ref[...]), 0.0)
        z2 = jnp.dot(z1, w2_ref[...], preferred_element_type=jnp.float32)
        z2 = jnp.maximum(_bn_rows(z2 + b2_ref[...], gg2_ref[...], tt2_ref[...]), 0.0)
        z3 = jnp.dot(z2, w3_ref[...], preferred_element_type=jnp.float32)
        o_ref[...] = jax.nn.sigmoid(z3 + b3_ref[...])

    return pl.pallas_call(
        body,
        out_shape=jax.ShapeDtypeStruct((n, a), jnp.float32),
    )(h1, agg, wn2, bnn2.reshape(1, h), g2.reshape(1, h), bt2.reshape(1, h),
      wa1, ba1.reshape(1, h), ga1.reshape(1, h), bta1.reshape(1, h),
      wa2, ba2.reshape(1, h), ga2.reshape(1, h), bta2.reshape(1, h),
      wa3, ba3.reshape(1, a))


def kernel(x, edge_index, edge_attr, batch, we1, bee1, wn1, bnn1, g1, bt1,
           we2, bee2, wn2, bnn2, g2, bt2, wa1, ba1, ga1, bta1,
           wa2, ba2, ga2, bta2, wa3, ba3):
    n, h = x.shape
    e = edge_attr.shape[0]
    nchb = -(-e // (_W * 8 * _CH))
    epad = _W * nchb * 8 * _CH
    npad = -(-n // (_NS * _CH)) * (_NS * _CH)

    src_p = jnp.concatenate(
        [edge_index[0].astype(jnp.int32),
         jnp.arange(epad - e, dtype=jnp.int32) % n])
    dst_p = jnp.concatenate(
        [edge_index[1].astype(jnp.int32),
         n + jnp.arange(epad - e, dtype=jnp.int32) % (npad - n)])
    src4 = src_p.reshape(_W, nchb, 8, _CH)
    dst4 = dst_p.reshape(_W, nchb, 8, _CH)
    ea_p = jnp.concatenate(
        [edge_attr, jnp.zeros((epad - e, edge_attr.shape[1]), jnp.float32)])

    ee1 = _edge_mlp(ea_p, we1, bee1).reshape(_W, nchb, 8, _CH, h)
    agg1 = _sc_edge_pass(x, src4, dst4, ee1, npad)
    ee2 = _edge_mlp(ea_p, we2, bee2).reshape(_W, nchb, 8, _CH, h)
    h1 = _node_update1(x, agg1, wn1, bnn1, g1, bt1)
    agg2 = _sc_edge_pass(h1, src4, dst4, ee2, npad)
    return _node_update2_head(h1, agg2, wn2, bnn2, g2, bt2,
                              wa1, ba1, ga1, bta1, wa2, ba2, ga2, bta2,
                              wa3, ba3)


# speedup vs baseline: 1.6140x; 1.0159x over previous
import functools

import jax
import jax.numpy as jnp
from jax import lax
from jax.experimental import pallas as pl
from jax.experimental.pallas import tpu as pltpu
from jax.experimental.pallas import tpu_sc as plsc

_NC = 2
_NS = 16
_L = 16
_W = _NC * _NS
_CH = 80


def _sc_edge_pass(xn, src4, dst4, ee5, npad):
    n, h = xn.shape
    _, nchb, _, ch = src4.shape
    rpt = npad // _NS
    nz = rpt // ch

    mesh = plsc.VectorSubcoreMesh(core_axis_name="c", subcore_axis_name="s")

    @functools.partial(
        pl.kernel,
        out_type=jax.ShapeDtypeStruct((_NC, npad, h), jnp.float32),
        mesh=mesh,
        scratch_types=[
            pltpu.VMEM((2, 8, ch), jnp.int32),
            pltpu.VMEM((2, 8, ch), jnp.int32),
            pltpu.VMEM((2, ch, h), jnp.float32),
            pltpu.VMEM((2, ch, h), jnp.float32),
            pltpu.VMEM_SHARED((npad, h), jnp.float32),
            pltpu.SemaphoreType.DMA,
            pltpu.SemaphoreType.DMA,
            pltpu.SemaphoreType.DMA,
            pltpu.SemaphoreType.DMA,
            pltpu.SemaphoreType.DMA,
            pltpu.SemaphoreType.DMA,
            pltpu.SemaphoreType.DMA,
        ],
    )
    def k(x_hbm, src_hbm, dst_hbm, ee_hbm, out_hbm,
          src_v, dst_v, xg_v, ee_v, agg_sh,
          sem_i, sem_g0, sem_g1, sem_e0, sem_e1, sem_s0, sem_s1):
        cid = lax.axis_index("c")
        sid = lax.axis_index("s")
        wid = cid * _NS + sid
        sem_g = (sem_g0, sem_g1)
        sem_e = (sem_e0, sem_e1)
        sem_s = (sem_s0, sem_s1)

        zero = jnp.zeros((_L,), jnp.float32)

        def zero_body(i, _):
            for j in range(h // _L):
                xg_v[0, i, pl.ds(j * _L, _L)] = zero
            return 0

        lax.fori_loop(0, ch, zero_body, 0)
        for r in range(nz):
            pltpu.sync_copy(xg_v.at[0],
                            agg_sh.at[pl.ds(sid * rpt + r * ch, ch)])
        plsc.subcore_barrier()

        def compute_chunk(p):
            def body(i, _):
                for j in range(h // _L):
                    s = pl.ds(j * _L, _L)
                    xg_v[p, i, s] = jnp.maximum(xg_v[p, i, s] + ee_v[p, i, s],
                                                0.0)
                return 0
            lax.fori_loop(0, ch, body, 0)

        def drain(dst_ref, sem):
            pltpu.make_async_copy(ee_hbm.at[wid, 0, 0], dst_ref, sem).wait()

        def emit_block(jb, q, first, last):
            if not last:
                ci_s = pltpu.async_copy(src_hbm.at[wid, jb + 1],
                                        src_v.at[1 - q], sem_i)
                ci_d = pltpu.async_copy(dst_hbm.at[wid, jb + 1],
                                        dst_v.at[1 - q], sem_i)
            for jj in range(8):
                p = jj & 1
                drain(xg_v.at[p], sem_g[p])
                drain(ee_v.at[p], sem_e[p])
                if not (first and jj == 0):
                    drain(xg_v.at[1 - p], sem_s[1 - p])
                if jj < 7:
                    pltpu.async_copy(x_hbm.at[src_v.at[q, jj + 1]],
                                     xg_v.at[1 - p], sem_g[1 - p])
                    pltpu.async_copy(ee_hbm.at[wid, jb, jj + 1],
                                     ee_v.at[1 - p], sem_e[1 - p])
                elif not last:
                    ci_s.wait()
                    ci_d.wait()
                    pltpu.async_copy(x_hbm.at[src_v.at[1 - q, 0]],
                                     xg_v.at[1 - p], sem_g[1 - p])
                    pltpu.async_copy(ee_hbm.at[wid, jb + 1, 0],
                                     ee_v.at[1 - p], sem_e[1 - p])
                compute_chunk(p)
                pltpu.async_copy(xg_v.at[p], agg_sh.at[dst_v.at[q, jj]],
                                 sem_s[p], add=True)
            return None

        pltpu.sync_copy(src_hbm.at[wid, 0], src_v.at[0])
        pltpu.sync_copy(dst_hbm.at[wid, 0], dst_v.at[0])
        pltpu.async_copy(x_hbm.at[src_v.at[0, 0]], xg_v.at[0], sem_g[0])
        pltpu.async_copy(ee_hbm.at[wid, 0, 0], ee_v.at[0], sem_e[0])

        emit_block(0, 0, True, False)

        def blk_body(jb, _):
            emit_block(jb, jb & 1, False, False)
            return 0

        lax.fori_loop(1, nchb - 1, blk_body, 0)
        emit_block(nchb - 1, (nchb - 1) & 1, False, True)
        drain(xg_v.at[1], sem_s[1])

        plsc.subcore_barrier()
        for r in range(nz):
            rows = pl.ds(sid * rpt + r * ch, ch)
            pltpu.sync_copy(agg_sh.at[rows], out_hbm.at[cid, rows])

    return k(xn, src4, dst4, ee5)


def _edge_mlp(ea, we, bee, epad):
    e, ed = ea.shape
    h = we.shape[1]
    be = 2560
    grid = epad // be

    def body(ea_ref, w_ref, b_ref, o_ref):
        o_ref[...] = jnp.dot(ea_ref[...], w_ref[...],
                             preferred_element_type=jnp.float32) + b_ref[...]

    return pl.pallas_call(
        body,
        grid=(grid,),
        in_specs=[
            pl.BlockSpec((be, ed), lambda i: (i, 0)),
            pl.BlockSpec((ed, h), lambda i: (0, 0)),
            pl.BlockSpec((1, h), lambda i: (0, 0)),
        ],
        out_specs=pl.BlockSpec((be, h), lambda i: (i, 0)),
        out_shape=jax.ShapeDtypeStruct((epad, h), jnp.float32),
    )(ea, we, bee.reshape(1, h))


def _bn_rows(hh, g, bt):
    m = jnp.mean(hh, axis=0, keepdims=True)
    v = jnp.mean((hh - m) ** 2, axis=0, keepdims=True)
    return (hh - m) / jnp.sqrt(v + 1e-5) * g + bt


def _node_update1(x, agg, wn, bnn, g, bt):
    n, h = x.shape

    def body(x_ref, a_ref, w_ref, b_ref, g_ref, t_ref, o_ref):
        t = x_ref[...] + a_ref[0, :n] + a_ref[1, :n]
        hh = jnp.dot(t, w_ref[...], preferred_element_type=jnp.float32)
        hh = _bn_rows(hh + b_ref[...], g_ref[...], t_ref[...])
        o_ref[...] = jnp.maximum(hh, 0.0)

    return pl.pallas_call(
        body,
        out_shape=jax.ShapeDtypeStruct((n, h), jnp.float32),
    )(x, agg, wn, bnn.reshape(1, h), g.reshape(1, h), bt.reshape(1, h))


def _node_update2_head(h1, agg, wn2, bnn2, g2, bt2,
                       wa1, ba1, ga1, bta1, wa2, ba2, ga2, bta2, wa3, ba3):
    n, h = h1.shape
    a = wa3.shape[1]

    def body(h1_ref, a_ref, wn_ref, bn_ref, g2_ref, t2_ref,
             w1_ref, b1_ref, g1_ref, t1_ref,
             w2_ref, b2_ref, gg2_ref, tt2_ref,
             w3_ref, b3_ref, o_ref):
        t = h1_ref[...] + a_ref[0, :n] + a_ref[1, :n]
        hh = jnp.dot(t, wn_ref[...], preferred_element_type=jnp.float32)
        hh = jnp.maximum(_bn_rows(hh + bn_ref[...], g2_ref[...], t2_ref[...]), 0.0)
        hh = jax.nn.sigmoid(hh)
        z1 = jnp.dot(hh, w1_ref[...], preferred_element_type=jnp.float32)
        z1 = jnp.maximum(_bn_rows(z1 + b1_ref[...], g1_ref[...], t1_ref[...]), 0.0)
        z2 = jnp.dot(z1, w2_ref[...], preferred_element_type=jnp.float32)
        z2 = jnp.maximum(_bn_rows(z2 + b2_ref[...], gg2_ref[...], tt2_ref[...]), 0.0)
        z3 = jnp.dot(z2, w3_ref[...], preferred_element_type=jnp.float32)
        o_ref[...] = jax.nn.sigmoid(z3 + b3_ref[...])

    return pl.pallas_call(
        body,
        out_shape=jax.ShapeDtypeStruct((n, a), jnp.float32),
    )(h1, agg, wn2, bnn2.reshape(1, h), g2.reshape(1, h), bt2.reshape(1, h),
      wa1, ba1.reshape(1, h), ga1.reshape(1, h), bta1.reshape(1, h),
      wa2, ba2.reshape(1, h), ga2.reshape(1, h), bta2.reshape(1, h),
      wa3, ba3.reshape(1, a))


def kernel(x, edge_index, edge_attr, batch, we1, bee1, wn1, bnn1, g1, bt1,
           we2, bee2, wn2, bnn2, g2, bt2, wa1, ba1, ga1, bta1,
           wa2, ba2, ga2, bta2, wa3, ba3):
    n, h = x.shape
    e = edge_attr.shape[0]
    nchb = -(-e // (_W * 8 * _CH))
    epad = _W * nchb * 8 * _CH
    npad = -(-n // (_NS * _CH)) * (_NS * _CH)

    src_p = jnp.concatenate(
        [edge_index[0].astype(jnp.int32),
         jnp.arange(epad - e, dtype=jnp.int32) % n])
    dst_p = jnp.concatenate(
        [edge_index[1].astype(jnp.int32),
         n + jnp.arange(epad - e, dtype=jnp.int32) % (npad - n)])
    src4 = src_p.reshape(_W, nchb, 8, _CH)
    dst4 = dst_p.reshape(_W, nchb, 8, _CH)
    ee1 = _edge_mlp(edge_attr, we1, bee1, epad).reshape(_W, nchb, 8, _CH, h)
    agg1 = _sc_edge_pass(x, src4, dst4, ee1, npad)
    ee2 = _edge_mlp(edge_attr, we2, bee2, epad).reshape(_W, nchb, 8, _CH, h)
    h1 = _node_update1(x, agg1, wn1, bnn1, g1, bt1)
    agg2 = _sc_edge_pass(h1, src4, dst4, ee2, npad)
    return _node_update2_head(h1, agg2, wn2, bnn2, g2, bt2,
                              wa1, ba1, ga1, bta1, wa2, ba2, ga2, bta2,
                              wa3, ba3)
